# Initial kernel scaffold; baseline (speedup 1.0000x reference)
import functools

import jax
import jax.numpy as jnp
from jax import lax
from jax.experimental import pallas as pl
from jax.experimental.pallas import tpu as pltpu
from jax.experimental.pallas import tpu_sc as plsc

BATCH = 16384
HIST = 200
EMBED_DIM = 32
N = BATCH * HIST

_info = plsc.get_sparse_core_info()
NC = _info.num_cores
NS = _info.num_subcores
NW = NC * NS

PER_W = N // NW
CHUNK = 128
N_CHUNKS = PER_W // CHUNK

_mesh = plsc.VectorSubcoreMesh(core_axis_name="c", subcore_axis_name="s")


@functools.partial(
    pl.kernel,
    mesh=_mesh,
    out_type=jax.ShapeDtypeStruct((N, EMBED_DIM), jnp.float32),
    scratch_types=[
        pltpu.VMEM((CHUNK,), jnp.int32),
        pltpu.VMEM((CHUNK, EMBED_DIM), jnp.float32),
        pltpu.SemaphoreType.DMA,
    ],
)
def _gather_kernel(idx_hbm, table_hbm, out_hbm, idx_v, rows_v, sem):
    wid = lax.axis_index("s") * NC + lax.axis_index("c")
    base = wid * PER_W

    def body(c, carry):
        start = base + c * CHUNK
        pltpu.sync_copy(idx_hbm.at[pl.ds(start, CHUNK)], idx_v)
        pltpu.async_copy(table_hbm.at[idx_v], rows_v, sem).wait()
        pltpu.sync_copy(rows_v, out_hbm.at[pl.ds(start, CHUNK)])
        return carry

    lax.fori_loop(0, N_CHUNKS, body, 0)


def kernel(x, weight):
    out = _gather_kernel(x.reshape(N), weight)
    return out.reshape(BATCH, HIST, EMBED_DIM)


# speedup vs baseline: 3.6446x; 3.6446x over previous
import functools

import jax
import jax.numpy as jnp
from jax import lax
from jax.experimental import pallas as pl
from jax.experimental.pallas import tpu as pltpu
from jax.experimental.pallas import tpu_sc as plsc

BATCH = 16384
HIST = 200
EMBED_DIM = 32
N = BATCH * HIST

_info = plsc.get_sparse_core_info()
NC = _info.num_cores
NS = _info.num_subcores
NW = NC * NS

PER_W = N // NW
CHUNK = 128
N_CHUNKS = PER_W // CHUNK

_mesh = plsc.VectorSubcoreMesh(core_axis_name="c", subcore_axis_name="s")


@functools.partial(
    pl.kernel,
    mesh=_mesh,
    compiler_params=pltpu.CompilerParams(use_tc_tiling_on_sc=False),
    out_type=jax.ShapeDtypeStruct((N, EMBED_DIM), jnp.float32),
    scratch_types=[
        pltpu.VMEM((CHUNK,), jnp.int32),
        pltpu.VMEM((CHUNK, EMBED_DIM), jnp.float32),
        pltpu.SemaphoreType.DMA,
    ],
)
def _gather_kernel(idx_hbm, table_hbm, out_hbm, idx_v, rows_v, sem):
    wid = lax.axis_index("s") * NC + lax.axis_index("c")
    base = wid * PER_W

    def body(c, carry):
        start = base + c * CHUNK
        pltpu.sync_copy(idx_hbm.at[pl.ds(start, CHUNK)], idx_v)
        pltpu.async_copy(table_hbm.at[idx_v], rows_v, sem).wait()
        pltpu.sync_copy(rows_v, out_hbm.at[pl.ds(start, CHUNK)])
        return carry

    lax.fori_loop(0, N_CHUNKS, body, 0)


def kernel(x, weight):
    out = _gather_kernel(x.reshape(N), weight)
    return out.reshape(BATCH, HIST, EMBED_DIM)


# speedup vs baseline: 5.0467x; 1.3847x over previous
import functools

import jax
import jax.numpy as jnp
from jax import lax
from jax.experimental import pallas as pl
from jax.experimental.pallas import tpu as pltpu
from jax.experimental.pallas import tpu_sc as plsc

BATCH = 16384
HIST = 200
EMBED_DIM = 32
N = BATCH * HIST

_info = plsc.get_sparse_core_info()
NC = _info.num_cores
NS = _info.num_subcores
NW = NC * NS

PER_W = N // NW
CHUNK = 1024
NBUF = 2
N_CHUNKS = PER_W // CHUNK
N_ROUNDS = N_CHUNKS // NBUF

_mesh = plsc.VectorSubcoreMesh(core_axis_name="c", subcore_axis_name="s")


@functools.partial(
    pl.kernel,
    mesh=_mesh,
    compiler_params=pltpu.CompilerParams(use_tc_tiling_on_sc=False),
    out_type=jax.ShapeDtypeStruct((N, EMBED_DIM), jnp.float32),
    scratch_types=(
        [pltpu.VMEM((CHUNK,), jnp.int32) for _ in range(NBUF)]
        + [pltpu.VMEM((CHUNK, EMBED_DIM), jnp.float32) for _ in range(NBUF)]
        + [pltpu.SemaphoreType.DMA for _ in range(2 * NBUF)]
    ),
)
def _gather_kernel(idx_hbm, table_hbm, out_hbm, *bufs):
    idx_v = bufs[:NBUF]
    rows_v = bufs[NBUF:2 * NBUF]
    gsem = bufs[2 * NBUF:3 * NBUF]
    wsem = bufs[3 * NBUF:4 * NBUF]

    wid = lax.axis_index("s") * NC + lax.axis_index("c")
    base = wid * PER_W

    for b in range(NBUF):
        pltpu.sync_copy(idx_hbm.at[pl.ds(base + b * CHUNK, CHUNK)], idx_v[b])
        pltpu.async_copy(table_hbm.at[idx_v[b]], rows_v[b], gsem[b])

    def body(r, carry):
        for b in range(NBUF):
            c = r * NBUF + b
            cn = c + NBUF
            start = base + c * CHUNK
            pltpu.make_async_copy(table_hbm.at[idx_v[b]], rows_v[b], gsem[b]).wait()
            pltpu.async_copy(rows_v[b], out_hbm.at[pl.ds(start, CHUNK)], wsem[b])
            pltpu.sync_copy(idx_hbm.at[pl.ds(base + cn * CHUNK, CHUNK)], idx_v[b])
            pltpu.make_async_copy(rows_v[b], out_hbm.at[pl.ds(start, CHUNK)], wsem[b]).wait()
            pltpu.async_copy(table_hbm.at[idx_v[b]], rows_v[b], gsem[b])
        return carry

    lax.fori_loop(0, N_ROUNDS - 1, body, 0)

    for b in range(NBUF):
        c = (N_ROUNDS - 1) * NBUF + b
        start = base + c * CHUNK
        pltpu.make_async_copy(table_hbm.at[idx_v[b]], rows_v[b], gsem[b]).wait()
        pltpu.async_copy(rows_v[b], out_hbm.at[pl.ds(start, CHUNK)], wsem[b])
    for b in range(NBUF):
        c = (N_ROUNDS - 1) * NBUF + b
        start = base + c * CHUNK
        pltpu.make_async_copy(rows_v[b], out_hbm.at[pl.ds(start, CHUNK)], wsem[b]).wait()


def kernel(x, weight):
    out = _gather_kernel(x.reshape(N), weight)
    return out.reshape(BATCH, HIST, EMBED_DIM)


# speedup vs baseline: 5.0523x; 1.0011x over previous
import functools

import jax
import jax.numpy as jnp
from jax import lax
from jax.experimental import pallas as pl
from jax.experimental.pallas import tpu as pltpu
from jax.experimental.pallas import tpu_sc as plsc

BATCH = 16384
HIST = 200
EMBED_DIM = 32
N = BATCH * HIST

_info = plsc.get_sparse_core_info()
NC = _info.num_cores
NS = _info.num_subcores
NW = NC * NS

PER_W = N // NW
CHUNK = 800
NBUF = 4
N_CHUNKS = PER_W // CHUNK
N_ROUNDS = N_CHUNKS // NBUF

_mesh = plsc.VectorSubcoreMesh(core_axis_name="c", subcore_axis_name="s")


@functools.partial(
    pl.kernel,
    mesh=_mesh,
    compiler_params=pltpu.CompilerParams(use_tc_tiling_on_sc=False),
    out_type=jax.ShapeDtypeStruct((N, EMBED_DIM), jnp.float32),
    scratch_types=(
        [pltpu.VMEM((CHUNK,), jnp.int32) for _ in range(NBUF)]
        + [pltpu.VMEM((CHUNK, EMBED_DIM), jnp.float32) for _ in range(NBUF)]
        + [pltpu.SemaphoreType.DMA for _ in range(2 * NBUF)]
    ),
)
def _gather_kernel(idx_hbm, table_hbm, out_hbm, *bufs):
    idx_v = bufs[:NBUF]
    rows_v = bufs[NBUF:2 * NBUF]
    gsem = bufs[2 * NBUF:3 * NBUF]
    wsem = bufs[3 * NBUF:4 * NBUF]

    wid = lax.axis_index("s") * NC + lax.axis_index("c")
    base = wid * PER_W

    for b in range(NBUF):
        pltpu.sync_copy(idx_hbm.at[pl.ds(base + b * CHUNK, CHUNK)], idx_v[b])
        pltpu.async_copy(table_hbm.at[idx_v[b]], rows_v[b], gsem[b])

    def body(r, carry):
        for b in range(NBUF):
            c = r * NBUF + b
            cn = c + NBUF
            start = base + c * CHUNK
            pltpu.make_async_copy(table_hbm.at[idx_v[b]], rows_v[b], gsem[b]).wait()
            pltpu.async_copy(rows_v[b], out_hbm.at[pl.ds(start, CHUNK)], wsem[b])
            pltpu.sync_copy(idx_hbm.at[pl.ds(base + cn * CHUNK, CHUNK)], idx_v[b])
            pltpu.make_async_copy(rows_v[b], out_hbm.at[pl.ds(start, CHUNK)], wsem[b]).wait()
            pltpu.async_copy(table_hbm.at[idx_v[b]], rows_v[b], gsem[b])
        return carry

    lax.fori_loop(0, N_ROUNDS - 1, body, 0)

    for b in range(NBUF):
        c = (N_ROUNDS - 1) * NBUF + b
        start = base + c * CHUNK
        pltpu.make_async_copy(table_hbm.at[idx_v[b]], rows_v[b], gsem[b]).wait()
        pltpu.async_copy(rows_v[b], out_hbm.at[pl.ds(start, CHUNK)], wsem[b])
    for b in range(NBUF):
        c = (N_ROUNDS - 1) * NBUF + b
        start = base + c * CHUNK
        pltpu.make_async_copy(rows_v[b], out_hbm.at[pl.ds(start, CHUNK)], wsem[b]).wait()


def kernel(x, weight):
    out = _gather_kernel(x.reshape(N), weight)
    return out.reshape(BATCH, HIST, EMBED_DIM)


# speedup vs baseline: 5.0531x; 1.0002x over previous
import functools

import jax
import jax.numpy as jnp
from jax import lax
from jax.experimental import pallas as pl
from jax.experimental.pallas import tpu as pltpu
from jax.experimental.pallas import tpu_sc as plsc

BATCH = 16384
HIST = 200
EMBED_DIM = 32
N = BATCH * HIST

_info = plsc.get_sparse_core_info()
NC = _info.num_cores
NS = _info.num_subcores
NW = NC * NS

PER_W = N // NW
CHUNK = 1600
NBUF = 2
N_CHUNKS = PER_W // CHUNK
N_ROUNDS = N_CHUNKS // NBUF

_mesh = plsc.VectorSubcoreMesh(core_axis_name="c", subcore_axis_name="s")


@functools.partial(
    pl.kernel,
    mesh=_mesh,
    compiler_params=pltpu.CompilerParams(use_tc_tiling_on_sc=False),
    out_type=jax.ShapeDtypeStruct((N, EMBED_DIM), jnp.float32),
    scratch_types=(
        [pltpu.VMEM((CHUNK,), jnp.int32) for _ in range(NBUF)]
        + [pltpu.VMEM((CHUNK, EMBED_DIM), jnp.float32) for _ in range(NBUF)]
        + [pltpu.SemaphoreType.DMA for _ in range(2 * NBUF)]
    ),
)
def _gather_kernel(idx_hbm, table_hbm, out_hbm, *bufs):
    idx_v = bufs[:NBUF]
    rows_v = bufs[NBUF:2 * NBUF]
    gsem = bufs[2 * NBUF:3 * NBUF]
    wsem = bufs[3 * NBUF:4 * NBUF]

    wid = lax.axis_index("s") * NC + lax.axis_index("c")
    base = wid * PER_W

    for b in range(NBUF):
        pltpu.sync_copy(idx_hbm.at[pl.ds(base + b * CHUNK, CHUNK)], idx_v[b])
        pltpu.async_copy(table_hbm.at[idx_v[b]], rows_v[b], gsem[b])

    def body(r, carry):
        for b in range(NBUF):
            c = r * NBUF + b
            cn = c + NBUF
            start = base + c * CHUNK
            pltpu.make_async_copy(table_hbm.at[idx_v[b]], rows_v[b], gsem[b]).wait()
            pltpu.async_copy(rows_v[b], out_hbm.at[pl.ds(start, CHUNK)], wsem[b])
            pltpu.sync_copy(idx_hbm.at[pl.ds(base + cn * CHUNK, CHUNK)], idx_v[b])
            pltpu.make_async_copy(rows_v[b], out_hbm.at[pl.ds(start, CHUNK)], wsem[b]).wait()
            pltpu.async_copy(table_hbm.at[idx_v[b]], rows_v[b], gsem[b])
        return carry

    lax.fori_loop(0, N_ROUNDS - 1, body, 0)

    for b in range(NBUF):
        c = (N_ROUNDS - 1) * NBUF + b
        start = base + c * CHUNK
        pltpu.make_async_copy(table_hbm.at[idx_v[b]], rows_v[b], gsem[b]).wait()
        pltpu.async_copy(rows_v[b], out_hbm.at[pl.ds(start, CHUNK)], wsem[b])
    for b in range(NBUF):
        c = (N_ROUNDS - 1) * NBUF + b
        start = base + c * CHUNK
        pltpu.make_async_copy(rows_v[b], out_hbm.at[pl.ds(start, CHUNK)], wsem[b]).wait()


def kernel(x, weight):
    out = _gather_kernel(x.reshape(N), weight)
    return out.reshape(BATCH, HIST, EMBED_DIM)
